# Initial kernel scaffold; baseline (speedup 1.0000x reference)
import jax
import jax.numpy as jnp
from jax.experimental import pallas as pl


def kernel(x, embed_weight, bow_bias):
    raise NotImplementedError("write your pallas kernel here")


# speedup vs baseline: 15.7461x; 15.7461x over previous
import functools

import jax
import jax.numpy as jnp
from jax import lax
from jax.experimental import pallas as pl
from jax.experimental.pallas import tpu as pltpu
from jax.experimental.pallas import tpu_sc as plsc


_NC = 2
_NS = 16
_NW = _NC * _NS
_LANES = 16

_CB = 8
_GSPLIT = 2


def _sum_block(rows_ref, out_ref, out_row0, n_rows, l_per_row):
    half = l_per_row // 2

    for i in range(n_rows):
        flat0 = i * l_per_row

        def body(t, accs):
            a0, a1, b0, b1 = accs
            r = flat0 + t * 2
            a0 = a0 + rows_ref[r, 0:16]
            a1 = a1 + rows_ref[r, 16:32]
            b0 = b0 + rows_ref[r + 1, 0:16]
            b1 = b1 + rows_ref[r + 1, 16:32]
            return a0, a1, b0, b1

        z = jnp.zeros((_LANES,), jnp.float32)
        a0, a1, b0, b1 = lax.fori_loop(0, half, body, (z, z, z, z))
        out_ref[out_row0 + i, 0:16] = a0 + b0
        out_ref[out_row0 + i, 16:32] = a1 + b1


def _make_sc_embed_sum(B, V, T, L):
    assert T == 32 and L % (2 * _GSPLIT) == 0
    b_per_w = B // _NW
    n_blocks = b_per_w // _CB
    g_len = L // _GSPLIT
    rows_per_block = _CB * L
    mesh = plsc.VectorSubcoreMesh(core_axis_name="c", subcore_axis_name="s")

    @functools.partial(
        pl.kernel,
        out_type=jax.ShapeDtypeStruct((B, T), jnp.float32),
        mesh=mesh,
        compiler_params=pltpu.CompilerParams(use_tc_tiling_on_sc=False),
        scratch_types=[
            pltpu.VMEM((2, _CB, _GSPLIT, g_len), jnp.int32),
            pltpu.VMEM((rows_per_block, T), jnp.float32),
            pltpu.VMEM((rows_per_block, T), jnp.float32),
            pltpu.VMEM((b_per_w, T), jnp.float32),
            pltpu.SemaphoreType.DMA,
            pltpu.SemaphoreType.DMA,
            pltpu.SemaphoreType.DMA,
            pltpu.SemaphoreType.DMA,
        ],
    )
    def sc_embed_sum(x_hbm, tab_hbm, out_hbm, idx_v, rows0, rows1, out_v,
                     gsem0, gsem1, isem0, isem1):
        wid = lax.axis_index("s") * _NC + lax.axis_index("c")
        base = wid * b_per_w
        rows_bufs = (rows0, rows1)
        gsems = (gsem0, gsem1)
        isems = (isem0, isem1)

        def idx_src(kb):
            return x_hbm.at[pl.ds(base + kb * _CB, _CB)]

        def fire_gathers(kb_buf, rows_ref, sem):
            for i in range(_CB):
                for j in range(_GSPLIT):
                    pltpu.async_copy(
                        tab_hbm.at[idx_v.at[kb_buf, i, j]],
                        rows_ref.at[pl.ds((i * _GSPLIT + j) * g_len, g_len)],
                        sem,
                    )

        def drain_gathers(rows_ref, sem):
            pltpu.make_async_copy(
                tab_hbm.at[pl.ds(0, rows_per_block)], rows_ref, sem
            ).wait()

        pltpu.sync_copy(idx_src(0), idx_v.at[0])
        fire_gathers(0, rows0, gsem0)
        pltpu.async_copy(idx_src(1), idx_v.at[1], isem1)

        def half_step(kb, cur):
            rows_c = rows_bufs[cur]
            rows_n = rows_bufs[1 - cur]
            drain_gathers(rows_c, gsems[cur])

            @pl.when(kb + 2 < n_blocks)
            def _():
                pltpu.async_copy(idx_src(kb + 2), idx_v.at[cur], isems[cur])

            @pl.when(kb + 1 < n_blocks)
            def _():
                pltpu.make_async_copy(
                    idx_src(kb + 1), idx_v.at[1 - cur], isems[1 - cur]
                ).wait()
                fire_gathers(1 - cur, rows_n, gsems[1 - cur])

            _sum_block(rows_c, out_v, kb * _CB, _CB, L)

        def body(t, carry):
            half_step(2 * t, 0)
            half_step(2 * t + 1, 1)
            return carry

        lax.fori_loop(0, n_blocks // 2, body, 0)
        pltpu.sync_copy(out_v, out_hbm.at[pl.ds(base, b_per_w)])

    return sc_embed_sum


def _logsoftmax_body(s_ref, b_ref, o_ref):
    s = s_ref[...] + b_ref[...]
    m = jnp.max(s, axis=-1, keepdims=True)
    e = jnp.exp(s - m)
    lse = jnp.log(jnp.sum(e, axis=-1, keepdims=True))
    o_ref[...] = (s - m) - lse


def _tc_log_softmax(scores, bias):
    B, T = scores.shape
    blk = min(2048, B)
    return pl.pallas_call(
        _logsoftmax_body,
        out_shape=jax.ShapeDtypeStruct((B, T), jnp.float32),
        grid=(B // blk,),
        in_specs=[
            pl.BlockSpec((blk, T), lambda i: (i, 0)),
            pl.BlockSpec((1, T), lambda i: (0, 0)),
        ],
        out_specs=pl.BlockSpec((blk, T), lambda i: (i, 0)),
    )(scores, bias.reshape(1, T))


def kernel(x, embed_weight, bow_bias):
    B, L = x.shape
    V, T = embed_weight.shape
    x3 = x.reshape(B, _GSPLIT, L // _GSPLIT)
    sc = _make_sc_embed_sum(B, V, T, L)
    scores = sc(x3, embed_weight)
    return _tc_log_softmax(scores, bow_bias)
